# Initial kernel scaffold; baseline (speedup 1.0000x reference)
import jax
import jax.numpy as jnp
from jax.experimental import pallas as pl


def kernel(x, edge_index, W1, a_src1, a_dst1, b1, W2, a_src2, a_dst2, b2, Wih_f, Whh_f, bih_f, bhh_f, Wih_b, Whh_b, bih_b, bhh_b, Watt, batt, Wn1, bn1, Wn2, bn2, We1, be1, We2, be2):
    raise NotImplementedError("write your pallas kernel here")


# speedup vs baseline: 1.7487x; 1.7487x over previous
import functools

import jax
import jax.numpy as jnp
from jax.experimental import pallas as pl
from jax.experimental.pallas import tpu as pltpu

N = 50000
E = 800000
C = 128
H = 128

NODE_BLK = 2048
N_PAD = ((N + NODE_BLK - 1) // NODE_BLK) * NODE_BLK
EDGE_BLK = 8192
E_PAD = ((E + EDGE_BLK - 1) // EDGE_BLK) * EDGE_BLK


def _heads_body(hjk_ref, wn1_ref, bn1_ref, wn2_ref, bn2_ref, node_ref):
    hjk = hjk_ref[...]
    t = jnp.maximum(hjk @ wn1_ref[...] + bn1_ref[...][None, :], 0.0)
    logits = t @ wn2_ref[...] + bn2_ref[...][None, :]
    m = jnp.max(logits, axis=1, keepdims=True)
    ex = jnp.exp(logits - m)
    node_ref[...] = ex / jnp.sum(ex, axis=1, keepdims=True)


def _node_head(hjk, Wn1, bn1, Wn2, bn2):
    hjk_p = jnp.pad(hjk, ((0, N_PAD - N), (0, 0)))
    grid = (N_PAD // NODE_BLK,)
    out = pl.pallas_call(
        _heads_body,
        grid=grid,
        in_specs=[
            pl.BlockSpec((NODE_BLK, C), lambda i: (i, 0)),
            pl.BlockSpec((C, C), lambda i: (0, 0)),
            pl.BlockSpec((C,), lambda i: (0,)),
            pl.BlockSpec((C, 2), lambda i: (0, 0)),
            pl.BlockSpec((2,), lambda i: (0,)),
        ],
        out_specs=pl.BlockSpec((NODE_BLK, 2), lambda i: (i, 0)),
        out_shape=jax.ShapeDtypeStruct((N_PAD, 2), jnp.float32),
    )(hjk_p, Wn1, bn1, Wn2, bn2)
    return out[:N]


def _edge_head_body(p_ref, q_ref, we2_ref, be2_ref, out_ref):
    t = jnp.maximum(p_ref[...] + q_ref[...], 0.0)
    logits = t @ we2_ref[...] + be2_ref[...][None, :]
    m = jnp.max(logits, axis=1, keepdims=True)
    ex = jnp.exp(logits - m)
    out_ref[...] = ex / jnp.sum(ex, axis=1, keepdims=True)


def _edge_head(p_rows, q_rows, We2, be2):
    p_rows = jnp.pad(p_rows, ((0, E_PAD - E), (0, 0)))
    q_rows = jnp.pad(q_rows, ((0, E_PAD - E), (0, 0)))
    grid = (E_PAD // EDGE_BLK,)
    out = pl.pallas_call(
        _edge_head_body,
        grid=grid,
        in_specs=[
            pl.BlockSpec((EDGE_BLK, C), lambda i: (i, 0)),
            pl.BlockSpec((EDGE_BLK, C), lambda i: (i, 0)),
            pl.BlockSpec((C, 2), lambda i: (0, 0)),
            pl.BlockSpec((2,), lambda i: (0,)),
        ],
        out_specs=pl.BlockSpec((EDGE_BLK, 2), lambda i: (i, 0)),
        out_shape=jax.ShapeDtypeStruct((E_PAD, 2), jnp.float32),
    )(p_rows, q_rows, We2, be2)
    return out[:E]


def _gat_layer(x, row, col, W, a_src, a_dst, b):
    h = x @ W
    s_src = h @ a_src
    s_dst = h @ a_dst
    e = s_src[col] + s_dst[row]
    e = jnp.where(e > 0, e, 0.2 * e)
    w = jnp.exp(e)
    e_self = s_src + s_dst
    e_self = jnp.where(e_self > 0, e_self, 0.2 * e_self)
    w_self = jnp.exp(e_self)
    num = jax.ops.segment_sum(h[col] * w[:, None], row, num_segments=N)
    num = num + h * w_self[:, None]
    den = jax.ops.segment_sum(w, row, num_segments=N) + w_self
    return num / (den + 1e-16)[:, None] + b


def _lstm_dir(xs, Wih, Whh, bih, bhh, reverse):
    n = xs[0].shape[0]
    h = jnp.zeros((n, H), jnp.float32)
    c = jnp.zeros((n, H), jnp.float32)
    outs = [None, None]
    steps = [1, 0] if reverse else [0, 1]
    for t in steps:
        g = xs[t] @ Wih.T + h @ Whh.T + bih + bhh
        i, f, gg, o = jnp.split(g, 4, axis=-1)
        i = jax.nn.sigmoid(i)
        f = jax.nn.sigmoid(f)
        gg = jnp.tanh(gg)
        o = jax.nn.sigmoid(o)
        c = f * c + i * gg
        h = o * jnp.tanh(c)
        outs[t] = h
    return outs


def kernel(x, edge_index, W1, a_src1, a_dst1, b1, W2, a_src2, a_dst2, b2,
           Wih_f, Whh_f, bih_f, bhh_f, Wih_b, Whh_b, bih_b, bhh_b,
           Watt, batt, Wn1, bn1, Wn2, bn2, We1, be1, We2, be2):
    row = edge_index[0]
    col = edge_index[1]
    x1 = jax.nn.relu(_gat_layer(x, row, col, W1, a_src1, a_dst1, b1))
    x2 = jax.nn.relu(_gat_layer(x1, row, col, W2, a_src2, a_dst2, b2))
    xs = [x1, x2]
    hf = _lstm_dir(xs, Wih_f, Whh_f, bih_f, bhh_f, False)
    hb = _lstm_dir(xs, Wih_b, Whh_b, bih_b, bhh_b, True)
    wf = Watt[:H, 0]
    wb = Watt[H:, 0]
    att0 = hf[0] @ wf + hb[0] @ wb + batt[0]
    att1 = hf[1] @ wf + hb[1] @ wb + batt[0]
    m = jnp.maximum(att0, att1)
    a0 = jnp.exp(att0 - m)
    a1 = jnp.exp(att1 - m)
    s = a0 + a1
    hjk = x1 * (a0 / s)[:, None] + x2 * (a1 / s)[:, None]
    node_out = _node_head(hjk, Wn1, bn1, Wn2, bn2)
    p = hjk @ We1[:C] + be1
    q = hjk @ We1[C:]
    edge_out = _edge_head(p[row], q[col], We2, be2)
    return (node_out, edge_out)


# speedup vs baseline: 2.0745x; 1.1863x over previous
import functools

import jax
import jax.numpy as jnp
from jax import lax
from jax.experimental import pallas as pl
from jax.experimental.pallas import tpu as pltpu
from jax.experimental.pallas import tpu_sc as plsc

N = 50000
E = 800000
C = 128
H = 128

NW = 32
EPW = E // NW
EB = 200
NBATCH = EPW // EB

@functools.cache
def _sc_mesh():
    return plsc.VectorSubcoreMesh(core_axis_name="c", subcore_axis_name="s")


CH = 12
NC = 4608
NR = NC // 16
SS = NR // 6
N_PAD = CH * NC
NPT = N_PAD // 16
E_PAD_E = 819200
EPW_W = E_PAD_E // NW
WB = 1024
AB = 256
ACC_ROWS = NC


def _edge_w_body(ssrc_hbm, sdst_hbm, row_hbm, col_hbm, w_hbm,
                 rb, cb, sv1, sv2, wb, sem):
    wid = lax.axis_index("s") * 2 + lax.axis_index("c")
    base = wid * EPW_W

    def batch(g, _):
        off = base + g * WB
        pltpu.sync_copy(row_hbm.at[pl.ds(off, WB)], rb)
        pltpu.sync_copy(col_hbm.at[pl.ds(off, WB)], cb)
        pltpu.async_copy(ssrc_hbm.at[cb], sv1, sem).wait()
        pltpu.async_copy(sdst_hbm.at[rb], sv2, sem).wait()

        def grp(t, _):
            sl = pl.ds(t * 16, 16)
            e = sv1[sl] + sv2[sl]
            e = jnp.where(e > 0, e, 0.2 * e)
            wb[sl] = jnp.exp(e)
            return _

        lax.fori_loop(0, WB // 16, grp, None)
        pltpu.sync_copy(wb, w_hbm.at[pl.ds(off, WB)])
        return _

    lax.fori_loop(0, EPW_W // WB, batch, None)


def _edge_w(s_src_pad, s_dst_pad, row_pad, col_pad):
    return pl.kernel(
        _edge_w_body,
        mesh=_sc_mesh(),
        out_type=jax.ShapeDtypeStruct((E_PAD_E,), jnp.float32),
        scratch_types=[
            pltpu.VMEM((WB,), jnp.int32),
            pltpu.VMEM((WB,), jnp.int32),
            pltpu.VMEM((WB,), jnp.float32),
            pltpu.VMEM((WB,), jnp.float32),
            pltpu.VMEM((WB,), jnp.float32),
            pltpu.SemaphoreType.DMA,
        ],
    )(s_src_pad, s_dst_pad, row_pad, col_pad)


def _den_body(w_hbm, row_hbm, den_hbm, rb, wv, zden, sden, den_sp, sem):
    core = lax.axis_index("c")
    sub = lax.axis_index("s")

    def zd(t, _):
        zden[pl.ds(t * 16, 16)] = jnp.zeros((16,), jnp.float32)
        return _

    lax.fori_loop(0, NPT // 4 // 16, zd, None)
    for i in range(4):
        pltpu.sync_copy(zden.at[pl.ds(0, NPT // 4)],
                        den_sp.at[pl.ds(sub * NPT + i * (NPT // 4), NPT // 4)])
    plsc.subcore_barrier()

    wid = sub * 2 + core
    base = wid * EPW_W

    def batch(g, _):
        off = base + g * WB
        pltpu.sync_copy(row_hbm.at[pl.ds(off, WB)], rb)
        pltpu.sync_copy(w_hbm.at[pl.ds(off, WB)], wv)
        pltpu.sync_copy(wv, den_sp.at[rb], add=True)
        return _

    lax.fori_loop(0, EPW_W // WB, batch, None)
    plsc.subcore_barrier()
    pltpu.sync_copy(den_sp.at[pl.ds(sub * NPT, NPT)], sden)
    pltpu.sync_copy(sden, den_hbm.at[core, pl.ds(sub * NPT, NPT)])


def _den_partials(w_pad, row_pad):
    return pl.kernel(
        _den_body,
        mesh=_sc_mesh(),
        out_type=jax.ShapeDtypeStruct((2, N_PAD), jnp.float32),
        scratch_types=[
            pltpu.VMEM((WB,), jnp.int32),
            pltpu.VMEM((WB,), jnp.float32),
            pltpu.VMEM((NPT // 4,), jnp.float32),
            pltpu.VMEM((NPT,), jnp.float32),
            pltpu.VMEM_SHARED((N_PAD,), jnp.float32),
            pltpu.SemaphoreType.DMA,
        ],
    )(w_pad, row_pad)


def _gat_acc_body(h_hbm, w_hbm, row_hbm, col_hbm, num_hbm,
                  rb, cb, wv, lidx, rows_v, zbuf, sbuf,
                  acc_sp, sem):
    core = lax.axis_index("c")
    sub = lax.axis_index("s")

    def zrow(r, _):
        for j in range(C // 16):
            zbuf[r, pl.ds(j * 16, 16)] = jnp.zeros((16,), jnp.float32)
        return _

    lax.fori_loop(0, SS, zrow, None)

    for p in range(CH // 2):
        chunk = core * (CH // 2) + p
        lo = chunk * NC

        for i in range(6):
            pltpu.sync_copy(zbuf, acc_sp.at[pl.ds(sub * NR + i * SS, SS)])
        plsc.subcore_barrier()

        EPA = E_PAD_E // 16

        def batch(g, _):
            off = sub * EPA + g * AB
            pltpu.sync_copy(row_hbm.at[pl.ds(off, AB)], rb)
            pltpu.sync_copy(col_hbm.at[pl.ds(off, AB)], cb)
            pltpu.sync_copy(w_hbm.at[pl.ds(off, AB)], wv)

            def grp(t, _):
                sl = pl.ds(t * 16, 16)
                local = rb[sl] - lo
                inb = (local >= 0) & (local < NC)
                w16 = jnp.where(inb, wv[sl], 0.0)
                lidx[sl] = jnp.where(inb, local, 0)
                wv[sl] = w16
                return _

            lax.fori_loop(0, AB // 16, grp, None)
            pltpu.async_copy(h_hbm.at[cb], rows_v, sem).wait()

            def scale(t, _):
                w16 = wv[pl.ds(t * 16, 16)]
                for k in range(16):
                    wr = w16[k]
                    for j in range(C // 16):
                        sl = pl.ds(j * 16, 16)
                        rows_v[t * 16 + k, sl] = rows_v[t * 16 + k, sl] * wr
                return _

            lax.fori_loop(0, AB // 16, scale, None)
            pltpu.sync_copy(rows_v, acc_sp.at[lidx], add=True)
            return _

        lax.fori_loop(0, EPA // AB, batch, None)
        plsc.subcore_barrier()

        for i in range(6):
            pltpu.sync_copy(acc_sp.at[pl.ds(sub * NR + i * SS, SS)], sbuf)
            pltpu.sync_copy(sbuf, num_hbm.at[pl.ds(lo + sub * NR + i * SS, SS)])
        plsc.subcore_barrier()


def _gat_acc(h, w_pad, row_pad, col_pad):
    return pl.kernel(
        _gat_acc_body,
        mesh=_sc_mesh(),
        out_type=jax.ShapeDtypeStruct((N_PAD, C), jnp.float32),
        scratch_types=[
            pltpu.VMEM((AB,), jnp.int32),
            pltpu.VMEM((AB,), jnp.int32),
            pltpu.VMEM((AB,), jnp.float32),
            pltpu.VMEM((AB,), jnp.int32),
            pltpu.VMEM((AB, C), jnp.float32),
            pltpu.VMEM((SS, C), jnp.float32),
            pltpu.VMEM((SS, C), jnp.float32),
            pltpu.VMEM_SHARED((ACC_ROWS, C), jnp.float32),
            pltpu.SemaphoreType.DMA,
        ],
    )(h, w_pad, row_pad, col_pad)


def _edge_gather_body(p_hbm, q_hbm, row_hbm, col_hbm, r_hbm,
                      idxr_v, idxc_v, prows_v, qrows_v, sem):
    wid = lax.axis_index("s") * 2 + lax.axis_index("c")
    base = wid * EPW

    def batch(g, _):
        off = base + g * EB
        pltpu.sync_copy(row_hbm.at[pl.ds(off, EB)], idxr_v)
        pltpu.sync_copy(col_hbm.at[pl.ds(off, EB)], idxc_v)
        pltpu.async_copy(p_hbm.at[idxr_v], prows_v, sem).wait()
        pltpu.async_copy(q_hbm.at[idxc_v], qrows_v, sem).wait()

        def addrow(r, _):
            for j in range(C // 16):
                sl = pl.ds(j * 16, 16)
                prows_v[r, sl] = prows_v[r, sl] + qrows_v[r, sl]
            return _

        lax.fori_loop(0, EB, addrow, None)
        pltpu.sync_copy(prows_v, r_hbm.at[pl.ds(off, EB)])
        return _

    lax.fori_loop(0, NBATCH, batch, None)


@functools.partial(jax.jit, static_argnames=())
def _edge_gather_add(p, q, row, col):
    return pl.kernel(
        _edge_gather_body,
        mesh=_sc_mesh(),
        out_type=jax.ShapeDtypeStruct((E, C), jnp.float32),
        scratch_types=[
            pltpu.VMEM((EB,), jnp.int32),
            pltpu.VMEM((EB,), jnp.int32),
            pltpu.VMEM((EB, C), jnp.float32),
            pltpu.VMEM((EB, C), jnp.float32),
            pltpu.SemaphoreType.DMA,
        ],
    )(p, q, row, col)

NODE_BLK = 2048
NH_PAD = ((N + NODE_BLK - 1) // NODE_BLK) * NODE_BLK
EDGE_BLK = 8000


def _heads_body(hjk_ref, wn1_ref, bn1_ref, wn2_ref, bn2_ref, node_ref):
    hjk = hjk_ref[...]
    t = jnp.maximum(hjk @ wn1_ref[...] + bn1_ref[...][None, :], 0.0)
    logits = t @ wn2_ref[...] + bn2_ref[...][None, :]
    m = jnp.max(logits, axis=1, keepdims=True)
    ex = jnp.exp(logits - m)
    node_ref[...] = ex / jnp.sum(ex, axis=1, keepdims=True)


def _node_head(hjk, Wn1, bn1, Wn2, bn2):
    hjk_p = jnp.pad(hjk, ((0, NH_PAD - N), (0, 0)))
    grid = (NH_PAD // NODE_BLK,)
    out = pl.pallas_call(
        _heads_body,
        grid=grid,
        in_specs=[
            pl.BlockSpec((NODE_BLK, C), lambda i: (i, 0)),
            pl.BlockSpec((C, C), lambda i: (0, 0)),
            pl.BlockSpec((C,), lambda i: (0,)),
            pl.BlockSpec((C, 2), lambda i: (0, 0)),
            pl.BlockSpec((2,), lambda i: (0,)),
        ],
        out_specs=pl.BlockSpec((NODE_BLK, 2), lambda i: (i, 0)),
        out_shape=jax.ShapeDtypeStruct((NH_PAD, 2), jnp.float32),
    )(hjk_p, Wn1, bn1, Wn2, bn2)
    return out[:N]


def _edge_head_body(r_ref, we2_ref, be2_ref, out_ref):
    t = jnp.maximum(r_ref[...], 0.0)
    logits = t @ we2_ref[...] + be2_ref[...][None, :]
    m = jnp.max(logits, axis=1, keepdims=True)
    ex = jnp.exp(logits - m)
    out_ref[...] = ex / jnp.sum(ex, axis=1, keepdims=True)


def _edge_head(r_rows, We2, be2):
    grid = (E // EDGE_BLK,)
    out = pl.pallas_call(
        _edge_head_body,
        grid=grid,
        in_specs=[
            pl.BlockSpec((EDGE_BLK, C), lambda i: (i, 0)),
            pl.BlockSpec((C, 2), lambda i: (0, 0)),
            pl.BlockSpec((2,), lambda i: (0,)),
        ],
        out_specs=pl.BlockSpec((EDGE_BLK, 2), lambda i: (i, 0)),
        out_shape=jax.ShapeDtypeStruct((E, 2), jnp.float32),
    )(r_rows, We2, be2)
    return out


def _gat_layer(x, row_pad, col_pad, W, a_src, a_dst, b):
    h = x @ W
    s_src = h @ a_src
    s_dst = h @ a_dst
    s_src_pad = jnp.pad(s_src, (0, N_PAD - N))
    s_dst_pad = jnp.pad(s_dst, (0, N_PAD - N))
    w = _edge_w(s_src_pad, s_dst_pad, row_pad, col_pad)
    w_pad = jnp.where(jnp.arange(E_PAD_E) < E, w, 0.0)
    num_pad = _gat_acc(h, w_pad, row_pad, col_pad)
    den_parts = _den_partials(w_pad, row_pad)
    num = num_pad[:N]
    den = den_parts[0, :N] + den_parts[1, :N]
    e_self = s_src + s_dst
    e_self = jnp.where(e_self > 0, e_self, 0.2 * e_self)
    w_self = jnp.exp(e_self)
    num = num + h * w_self[:, None]
    den = den + w_self
    return num / (den + 1e-16)[:, None] + b


def _lstm_dir(xs, Wih, Whh, bih, bhh, reverse):
    n = xs[0].shape[0]
    h = jnp.zeros((n, H), jnp.float32)
    c = jnp.zeros((n, H), jnp.float32)
    outs = [None, None]
    steps = [1, 0] if reverse else [0, 1]
    for t in steps:
        g = xs[t] @ Wih.T + h @ Whh.T + bih + bhh
        i, f, gg, o = jnp.split(g, 4, axis=-1)
        i = jax.nn.sigmoid(i)
        f = jax.nn.sigmoid(f)
        gg = jnp.tanh(gg)
        o = jax.nn.sigmoid(o)
        c = f * c + i * gg
        h = o * jnp.tanh(c)
        outs[t] = h
    return outs


def kernel(x, edge_index, W1, a_src1, a_dst1, b1, W2, a_src2, a_dst2, b2,
           Wih_f, Whh_f, bih_f, bhh_f, Wih_b, Whh_b, bih_b, bhh_b,
           Watt, batt, Wn1, bn1, Wn2, bn2, We1, be1, We2, be2):
    row = edge_index[0]
    col = edge_index[1]
    row_pad = jnp.pad(row, (0, E_PAD_E - E))
    col_pad = jnp.pad(col, (0, E_PAD_E - E))
    x1 = jax.nn.relu(_gat_layer(x, row_pad, col_pad, W1, a_src1, a_dst1, b1))
    x2 = jax.nn.relu(_gat_layer(x1, row_pad, col_pad, W2, a_src2, a_dst2, b2))
    xs = [x1, x2]
    hf = _lstm_dir(xs, Wih_f, Whh_f, bih_f, bhh_f, False)
    hb = _lstm_dir(xs, Wih_b, Whh_b, bih_b, bhh_b, True)
    wf = Watt[:H, 0]
    wb = Watt[H:, 0]
    att0 = hf[0] @ wf + hb[0] @ wb + batt[0]
    att1 = hf[1] @ wf + hb[1] @ wb + batt[0]
    m = jnp.maximum(att0, att1)
    a0 = jnp.exp(att0 - m)
    a1 = jnp.exp(att1 - m)
    s = a0 + a1
    hjk = x1 * (a0 / s)[:, None] + x2 * (a1 / s)[:, None]
    node_out = _node_head(hjk, Wn1, bn1, Wn2, bn2)
    p = hjk @ We1[:C] + be1
    q = hjk @ We1[C:]
    r_rows = _edge_gather_add(p, q, row, col)
    edge_out = _edge_head(r_rows, We2, be2)
    return (node_out, edge_out)


# speedup vs baseline: 4.8712x; 2.3481x over previous
import functools

import jax
import jax.numpy as jnp
from jax import lax
from jax.experimental import pallas as pl
from jax.experimental.pallas import tpu as pltpu
from jax.experimental.pallas import tpu_sc as plsc

N = 50000
E = 800000
C = 128
H = 128

NW = 32
EPW = E // NW
EB = 200
NBATCH = EPW // EB

@functools.cache
def _sc_mesh():
    return plsc.VectorSubcoreMesh(core_axis_name="c", subcore_axis_name="s")


CH = 4
NC = 12800
NR = NC // 16
NSTR = 25
SS = NR // NSTR
N_PAD = CH * NC
NPT = N_PAD // 16
E_PAD_E = 819200
EPW_W = E_PAD_E // NW
WB = 1024
AB = 128
ACC_ROWS = NC


def _edge_w_body(ssrc_hbm, sdst_hbm, row_hbm, col_hbm, w_hbm,
                 rb, cb, sv1, sv2, wb, sem):
    wid = lax.axis_index("s") * 2 + lax.axis_index("c")
    base = wid * EPW_W

    def batch(g, _):
        off = base + g * WB
        pltpu.sync_copy(row_hbm.at[pl.ds(off, WB)], rb)
        pltpu.sync_copy(col_hbm.at[pl.ds(off, WB)], cb)
        pltpu.async_copy(ssrc_hbm.at[cb], sv1, sem).wait()
        pltpu.async_copy(sdst_hbm.at[rb], sv2, sem).wait()

        def grp(t, _):
            sl = pl.ds(t * 16, 16)
            e = sv1[sl] + sv2[sl]
            e = jnp.where(e > 0, e, 0.2 * e)
            wb[sl] = jnp.exp(e)
            return _

        lax.fori_loop(0, WB // 16, grp, None)
        pltpu.sync_copy(wb, w_hbm.at[pl.ds(off, WB)])
        return _

    lax.fori_loop(0, EPW_W // WB, batch, None)


def _edge_w(s_src_pad, s_dst_pad, row_pad, col_pad):
    return pl.kernel(
        _edge_w_body,
        mesh=_sc_mesh(),
        out_type=jax.ShapeDtypeStruct((E_PAD_E,), jnp.float32),
        scratch_types=[
            pltpu.VMEM((WB,), jnp.int32),
            pltpu.VMEM((WB,), jnp.int32),
            pltpu.VMEM((WB,), jnp.float32),
            pltpu.VMEM((WB,), jnp.float32),
            pltpu.VMEM((WB,), jnp.float32),
            pltpu.SemaphoreType.DMA,
        ],
    )(s_src_pad, s_dst_pad, row_pad, col_pad)


def _den_body(w_hbm, row_hbm, den_hbm, rb, wv, zden, sden, den_sp, sem):
    core = lax.axis_index("c")
    sub = lax.axis_index("s")

    def zd(t, _):
        zden[pl.ds(t * 16, 16)] = jnp.zeros((16,), jnp.float32)
        return _

    lax.fori_loop(0, NPT // 4 // 16, zd, None)
    for i in range(4):
        pltpu.sync_copy(zden.at[pl.ds(0, NPT // 4)],
                        den_sp.at[pl.ds(sub * NPT + i * (NPT // 4), NPT // 4)])
    plsc.subcore_barrier()

    wid = sub * 2 + core
    base = wid * EPW_W

    def batch(g, _):
        off = base + g * WB
        pltpu.sync_copy(row_hbm.at[pl.ds(off, WB)], rb)
        pltpu.sync_copy(w_hbm.at[pl.ds(off, WB)], wv)
        pltpu.sync_copy(wv, den_sp.at[rb], add=True)
        return _

    lax.fori_loop(0, EPW_W // WB, batch, None)
    plsc.subcore_barrier()
    pltpu.sync_copy(den_sp.at[pl.ds(sub * NPT, NPT)], sden)
    pltpu.sync_copy(sden, den_hbm.at[core, pl.ds(sub * NPT, NPT)])


def _den_partials(w_pad, row_pad):
    return pl.kernel(
        _den_body,
        mesh=_sc_mesh(),
        out_type=jax.ShapeDtypeStruct((2, N_PAD), jnp.float32),
        scratch_types=[
            pltpu.VMEM((WB,), jnp.int32),
            pltpu.VMEM((WB,), jnp.float32),
            pltpu.VMEM((NPT // 4,), jnp.float32),
            pltpu.VMEM((NPT,), jnp.float32),
            pltpu.VMEM_SHARED((N_PAD,), jnp.float32),
            pltpu.SemaphoreType.DMA,
        ],
    )(w_pad, row_pad)


def _gat_acc_body(h_hbm, w_hbm, row_hbm, col_hbm, num_hbm,
                  rb, cb, wv, lidx, rows_v, zbuf, sbuf,
                  acc_sp, sem):
    core = lax.axis_index("c")
    sub = lax.axis_index("s")

    def zrow(r, _):
        for j in range(C // 16):
            zbuf[r, pl.ds(j * 16, 16)] = jnp.zeros((16,), jnp.float32)
        return _

    lax.fori_loop(0, SS, zrow, None)

    for p in range(CH // 2):
        chunk = core * (CH // 2) + p
        lo = chunk * NC

        for i in range(NSTR):
            pltpu.sync_copy(zbuf, acc_sp.at[pl.ds(sub * NR + i * SS, SS)])
        plsc.subcore_barrier()

        EPA = E_PAD_E // 16

        def batch(g, _):
            off = sub * EPA + g * AB
            pltpu.sync_copy(row_hbm.at[pl.ds(off, AB)], rb)
            pltpu.sync_copy(col_hbm.at[pl.ds(off, AB)], cb)
            pltpu.sync_copy(w_hbm.at[pl.ds(off, AB)], wv)

            def grp(t, _):
                sl = pl.ds(t * 16, 16)
                local = rb[sl] - lo
                inb = (local >= 0) & (local < NC)
                w16 = jnp.where(inb, wv[sl], 0.0)
                lidx[sl] = jnp.where(inb, local, 0)
                wv[sl] = w16
                return _

            lax.fori_loop(0, AB // 16, grp, None)
            pltpu.async_copy(h_hbm.at[cb], rows_v, sem).wait()

            def scale(t, _):
                w16 = wv[pl.ds(t * 16, 16)]
                for k in range(16):
                    wr = w16[k]
                    for j in range(C // 16):
                        sl = pl.ds(j * 16, 16)
                        rows_v[t * 16 + k, sl] = rows_v[t * 16 + k, sl] * wr
                return _

            lax.fori_loop(0, AB // 16, scale, None)
            pltpu.sync_copy(rows_v, acc_sp.at[lidx], add=True)
            return _

        lax.fori_loop(0, EPA // AB, batch, None)
        plsc.subcore_barrier()

        for i in range(NSTR):
            pltpu.sync_copy(acc_sp.at[pl.ds(sub * NR + i * SS, SS)], sbuf)
            pltpu.sync_copy(sbuf, num_hbm.at[pl.ds(lo + sub * NR + i * SS, SS)])
        plsc.subcore_barrier()


def _gat_acc(h, w_pad, row_pad, col_pad):
    return pl.kernel(
        _gat_acc_body,
        mesh=_sc_mesh(),
        out_type=jax.ShapeDtypeStruct((N_PAD, C), jnp.float32),
        scratch_types=[
            pltpu.VMEM((AB,), jnp.int32),
            pltpu.VMEM((AB,), jnp.int32),
            pltpu.VMEM((AB,), jnp.float32),
            pltpu.VMEM((AB,), jnp.int32),
            pltpu.VMEM((AB, C), jnp.float32),
            pltpu.VMEM((SS, C), jnp.float32),
            pltpu.VMEM((SS, C), jnp.float32),
            pltpu.VMEM_SHARED((ACC_ROWS, C), jnp.float32),
            pltpu.SemaphoreType.DMA,
        ],
    )(h, w_pad, row_pad, col_pad)


def _edge_gather_body(p_hbm, q_hbm, row_hbm, col_hbm, r_hbm,
                      idxr_v, idxc_v, prows_v, qrows_v, sem):
    wid = lax.axis_index("s") * 2 + lax.axis_index("c")
    base = wid * EPW

    def batch(g, _):
        off = base + g * EB
        pltpu.sync_copy(row_hbm.at[pl.ds(off, EB)], idxr_v)
        pltpu.sync_copy(col_hbm.at[pl.ds(off, EB)], idxc_v)
        pltpu.async_copy(p_hbm.at[idxr_v], prows_v, sem).wait()
        pltpu.async_copy(q_hbm.at[idxc_v], qrows_v, sem).wait()

        def addrow(r, _):
            for j in range(C // 16):
                sl = pl.ds(j * 16, 16)
                prows_v[r, sl] = prows_v[r, sl] + qrows_v[r, sl]
            return _

        lax.fori_loop(0, EB, addrow, None)
        pltpu.sync_copy(prows_v, r_hbm.at[pl.ds(off, EB)])
        return _

    lax.fori_loop(0, NBATCH, batch, None)


@functools.partial(jax.jit, static_argnames=())
def _edge_gather_add(p, q, row, col):
    return pl.kernel(
        _edge_gather_body,
        mesh=_sc_mesh(),
        out_type=jax.ShapeDtypeStruct((E, C), jnp.float32),
        scratch_types=[
            pltpu.VMEM((EB,), jnp.int32),
            pltpu.VMEM((EB,), jnp.int32),
            pltpu.VMEM((EB, C), jnp.float32),
            pltpu.VMEM((EB, C), jnp.float32),
            pltpu.SemaphoreType.DMA,
        ],
    )(p, q, row, col)

NODE_BLK = 2048
NH_PAD = ((N + NODE_BLK - 1) // NODE_BLK) * NODE_BLK
EDGE_BLK = 8000


def _heads_body(hjk_ref, wn1_ref, bn1_ref, wn2_ref, bn2_ref, node_ref):
    hjk = hjk_ref[...]
    t = jnp.maximum(hjk @ wn1_ref[...] + bn1_ref[...][None, :], 0.0)
    logits = t @ wn2_ref[...] + bn2_ref[...][None, :]
    m = jnp.max(logits, axis=1, keepdims=True)
    ex = jnp.exp(logits - m)
    node_ref[...] = ex / jnp.sum(ex, axis=1, keepdims=True)


def _node_head(hjk, Wn1, bn1, Wn2, bn2):
    hjk_p = jnp.pad(hjk, ((0, NH_PAD - N), (0, 0)))
    grid = (NH_PAD // NODE_BLK,)
    out = pl.pallas_call(
        _heads_body,
        grid=grid,
        in_specs=[
            pl.BlockSpec((NODE_BLK, C), lambda i: (i, 0)),
            pl.BlockSpec((C, C), lambda i: (0, 0)),
            pl.BlockSpec((C,), lambda i: (0,)),
            pl.BlockSpec((C, 2), lambda i: (0, 0)),
            pl.BlockSpec((2,), lambda i: (0,)),
        ],
        out_specs=pl.BlockSpec((NODE_BLK, 2), lambda i: (i, 0)),
        out_shape=jax.ShapeDtypeStruct((NH_PAD, 2), jnp.float32),
    )(hjk_p, Wn1, bn1, Wn2, bn2)
    return out[:N]


def _edge_head_body(r_ref, we2_ref, be2_ref, out_ref):
    t = jnp.maximum(r_ref[...], 0.0)
    logits = t @ we2_ref[...] + be2_ref[...][None, :]
    m = jnp.max(logits, axis=1, keepdims=True)
    ex = jnp.exp(logits - m)
    out_ref[...] = ex / jnp.sum(ex, axis=1, keepdims=True)


def _edge_head(r_rows, We2, be2):
    grid = (E // EDGE_BLK,)
    out = pl.pallas_call(
        _edge_head_body,
        grid=grid,
        in_specs=[
            pl.BlockSpec((EDGE_BLK, C), lambda i: (i, 0)),
            pl.BlockSpec((C, 2), lambda i: (0, 0)),
            pl.BlockSpec((2,), lambda i: (0,)),
        ],
        out_specs=pl.BlockSpec((EDGE_BLK, 2), lambda i: (i, 0)),
        out_shape=jax.ShapeDtypeStruct((E, 2), jnp.float32),
    )(r_rows, We2, be2)
    return out


def _gat_layer(x, row_pad, col_pad, W, a_src, a_dst, b):
    h = x @ W
    s_src = h @ a_src
    s_dst = h @ a_dst
    s_src_pad = jnp.pad(s_src, (0, N_PAD - N))
    s_dst_pad = jnp.pad(s_dst, (0, N_PAD - N))
    w = _edge_w(s_src_pad, s_dst_pad, row_pad, col_pad)
    w_pad = jnp.where(jnp.arange(E_PAD_E) < E, w, 0.0)
    num_pad = _gat_acc(h, w_pad, row_pad, col_pad)
    den_parts = _den_partials(w_pad, row_pad)
    num = num_pad[:N]
    den = den_parts[0, :N] + den_parts[1, :N]
    e_self = s_src + s_dst
    e_self = jnp.where(e_self > 0, e_self, 0.2 * e_self)
    w_self = jnp.exp(e_self)
    num = num + h * w_self[:, None]
    den = den + w_self
    return num / (den + 1e-16)[:, None] + b


def _lstm_dir(xs, Wih, Whh, bih, bhh, reverse):
    n = xs[0].shape[0]
    h = jnp.zeros((n, H), jnp.float32)
    c = jnp.zeros((n, H), jnp.float32)
    outs = [None, None]
    steps = [1, 0] if reverse else [0, 1]
    for t in steps:
        g = xs[t] @ Wih.T + h @ Whh.T + bih + bhh
        i, f, gg, o = jnp.split(g, 4, axis=-1)
        i = jax.nn.sigmoid(i)
        f = jax.nn.sigmoid(f)
        gg = jnp.tanh(gg)
        o = jax.nn.sigmoid(o)
        c = f * c + i * gg
        h = o * jnp.tanh(c)
        outs[t] = h
    return outs


def kernel(x, edge_index, W1, a_src1, a_dst1, b1, W2, a_src2, a_dst2, b2,
           Wih_f, Whh_f, bih_f, bhh_f, Wih_b, Whh_b, bih_b, bhh_b,
           Watt, batt, Wn1, bn1, Wn2, bn2, We1, be1, We2, be2):
    row = edge_index[0]
    col = edge_index[1]
    row_pad = jnp.pad(row, (0, E_PAD_E - E))
    col_pad = jnp.pad(col, (0, E_PAD_E - E))
    x1 = jax.nn.relu(_gat_layer(x, row_pad, col_pad, W1, a_src1, a_dst1, b1))
    x2 = jax.nn.relu(_gat_layer(x1, row_pad, col_pad, W2, a_src2, a_dst2, b2))
    xs = [x1, x2]
    hf = _lstm_dir(xs, Wih_f, Whh_f, bih_f, bhh_f, False)
    hb = _lstm_dir(xs, Wih_b, Whh_b, bih_b, bhh_b, True)
    wf = Watt[:H, 0]
    wb = Watt[H:, 0]
    att0 = hf[0] @ wf + hb[0] @ wb + batt[0]
    att1 = hf[1] @ wf + hb[1] @ wb + batt[0]
    m = jnp.maximum(att0, att1)
    a0 = jnp.exp(att0 - m)
    a1 = jnp.exp(att1 - m)
    s = a0 + a1
    hjk = x1 * (a0 / s)[:, None] + x2 * (a1 / s)[:, None]
    node_out = _node_head(hjk, Wn1, bn1, Wn2, bn2)
    p = hjk @ We1[:C] + be1
    q = hjk @ We1[C:]
    r_rows = _edge_gather_add(p, q, row, col)
    edge_out = _edge_head(r_rows, We2, be2)
    return (node_out, edge_out)


# speedup vs baseline: 5.2327x; 1.0742x over previous
import functools

import jax
import jax.numpy as jnp
from jax import lax
from jax.experimental import pallas as pl
from jax.experimental.pallas import tpu as pltpu
from jax.experimental.pallas import tpu_sc as plsc

N = 50000
E = 800000
C = 128
H = 128

NW = 32
EPW = E // NW
EB = 200
NBATCH = EPW // EB

@functools.cache
def _sc_mesh():
    return plsc.VectorSubcoreMesh(core_axis_name="c", subcore_axis_name="s")


CH = 4
NC = 12544
NR = NC // 16
NSTR = 49
SS = NR // NSTR
ACC_PAD = CH * NC
N_PAD = 51200
NPT = N_PAD // 16
E_PAD_E = 819200
EPW_W = E_PAD_E // NW
WB = 1024
AB = 80
ACC_ROWS = NC


def _edge_w_body(ssrc_hbm, sdst_hbm, row_hbm, col_hbm, w_hbm,
                 rb, cb, sv1, sv2, wb, sem):
    wid = lax.axis_index("s") * 2 + lax.axis_index("c")
    base = wid * EPW_W

    def batch(g, _):
        off = base + g * WB
        pltpu.sync_copy(row_hbm.at[pl.ds(off, WB)], rb)
        pltpu.sync_copy(col_hbm.at[pl.ds(off, WB)], cb)
        pltpu.async_copy(ssrc_hbm.at[cb], sv1, sem).wait()
        pltpu.async_copy(sdst_hbm.at[rb], sv2, sem).wait()

        def grp(t, _):
            sl = pl.ds(t * 16, 16)
            e = sv1[sl] + sv2[sl]
            e = jnp.where(e > 0, e, 0.2 * e)
            wb[sl] = jnp.exp(e)
            return _

        lax.fori_loop(0, WB // 16, grp, None)
        pltpu.sync_copy(wb, w_hbm.at[pl.ds(off, WB)])
        return _

    lax.fori_loop(0, EPW_W // WB, batch, None)


def _edge_w(s_src_pad, s_dst_pad, row_pad, col_pad):
    return pl.kernel(
        _edge_w_body,
        mesh=_sc_mesh(),
        out_type=jax.ShapeDtypeStruct((E_PAD_E,), jnp.float32),
        scratch_types=[
            pltpu.VMEM((WB,), jnp.int32),
            pltpu.VMEM((WB,), jnp.int32),
            pltpu.VMEM((WB,), jnp.float32),
            pltpu.VMEM((WB,), jnp.float32),
            pltpu.VMEM((WB,), jnp.float32),
            pltpu.SemaphoreType.DMA,
        ],
    )(s_src_pad, s_dst_pad, row_pad, col_pad)


def _den_body(w_hbm, row_hbm, den_hbm, rb, wv, zden, sden, den_sp, sem):
    core = lax.axis_index("c")
    sub = lax.axis_index("s")

    def zd(t, _):
        zden[pl.ds(t * 16, 16)] = jnp.zeros((16,), jnp.float32)
        return _

    lax.fori_loop(0, NPT // 4 // 16, zd, None)
    for i in range(4):
        pltpu.sync_copy(zden.at[pl.ds(0, NPT // 4)],
                        den_sp.at[pl.ds(sub * NPT + i * (NPT // 4), NPT // 4)])
    plsc.subcore_barrier()

    wid = sub * 2 + core
    base = wid * EPW_W

    def batch(g, _):
        off = base + g * WB
        pltpu.sync_copy(row_hbm.at[pl.ds(off, WB)], rb)
        pltpu.sync_copy(w_hbm.at[pl.ds(off, WB)], wv)
        pltpu.sync_copy(wv, den_sp.at[rb], add=True)
        return _

    lax.fori_loop(0, EPW_W // WB, batch, None)
    plsc.subcore_barrier()
    pltpu.sync_copy(den_sp.at[pl.ds(sub * NPT, NPT)], sden)
    pltpu.sync_copy(sden, den_hbm.at[core, pl.ds(sub * NPT, NPT)])


def _den_partials(w_pad, row_pad):
    return pl.kernel(
        _den_body,
        mesh=_sc_mesh(),
        out_type=jax.ShapeDtypeStruct((2, N_PAD), jnp.float32),
        scratch_types=[
            pltpu.VMEM((WB,), jnp.int32),
            pltpu.VMEM((WB,), jnp.float32),
            pltpu.VMEM((NPT // 4,), jnp.float32),
            pltpu.VMEM((NPT,), jnp.float32),
            pltpu.VMEM_SHARED((N_PAD,), jnp.float32),
            pltpu.SemaphoreType.DMA,
        ],
    )(w_pad, row_pad)


def _gat_acc_body(h_hbm, w_hbm, row_hbm, col_hbm, num_hbm,
                  rb0, cb0, wv0, lidx0, rows0,
                  rb1, cb1, wv1, lidx1, rows1,
                  zbuf, sbuf, acc_sp, sem0, sem1):
    core = lax.axis_index("c")
    sub = lax.axis_index("s")

    def zrow(r, _):
        for j in range(C // 16):
            zbuf[r, pl.ds(j * 16, 16)] = jnp.zeros((16,), jnp.float32)
        return _

    lax.fori_loop(0, SS, zrow, None)

    EPA = E_PAD_E // 16
    NB = EPA // AB

    for p in range(CH // 2):
        chunk = core * (CH // 2) + p
        lo = chunk * NC

        for i in range(NSTR):
            pltpu.sync_copy(zbuf, acc_sp.at[pl.ds(sub * NR + i * SS, SS)])
        plsc.subcore_barrier()

        def issue(g, rb, cb, wv, lidx, sem, rows):
            off = sub * EPA + g * AB
            pltpu.sync_copy(row_hbm.at[pl.ds(off, AB)], rb)
            pltpu.sync_copy(col_hbm.at[pl.ds(off, AB)], cb)
            pltpu.sync_copy(w_hbm.at[pl.ds(off, AB)], wv)
            pltpu.async_copy(h_hbm.at[cb], rows, sem)

            def grp(t, _):
                sl = pl.ds(t * 16, 16)
                local = rb[sl] - lo
                inb = (local >= 0) & (local < NC)
                lidx[sl] = jnp.where(inb, local, 0)
                wv[sl] = jnp.where(inb, wv[sl], 0.0)
                return _

            lax.fori_loop(0, AB // 16, grp, None)

        def complete(cb, wv, lidx, sem, rows):
            pltpu.make_async_copy(h_hbm.at[pl.ds(0, AB)], rows, sem).wait()

            def scale(t, _):
                w16 = wv[pl.ds(t * 16, 16)]
                for k in range(16):
                    wr = w16[k]
                    for j in range(C // 16):
                        sl = pl.ds(j * 16, 16)
                        rows[t * 16 + k, sl] = rows[t * 16 + k, sl] * wr
                return _

            lax.fori_loop(0, AB // 16, scale, None)
            pltpu.sync_copy(rows, acc_sp.at[lidx], add=True)

        issue(0, rb0, cb0, wv0, lidx0, sem0, rows0)

        def pair(gg, _):
            g = gg * 2
            issue(g + 1, rb1, cb1, wv1, lidx1, sem1, rows1)
            complete(cb0, wv0, lidx0, sem0, rows0)
            issue(g + 2, rb0, cb0, wv0, lidx0, sem0, rows0)
            complete(cb1, wv1, lidx1, sem1, rows1)
            return _

        lax.fori_loop(0, NB // 2 - 1, pair, None)
        issue(NB - 1, rb1, cb1, wv1, lidx1, sem1, rows1)
        complete(cb0, wv0, lidx0, sem0, rows0)
        complete(cb1, wv1, lidx1, sem1, rows1)
        plsc.subcore_barrier()

        for i in range(NSTR):
            pltpu.sync_copy(acc_sp.at[pl.ds(sub * NR + i * SS, SS)], sbuf)
            pltpu.sync_copy(sbuf, num_hbm.at[pl.ds(lo + sub * NR + i * SS, SS)])
        plsc.subcore_barrier()


def _gat_acc(h, w_pad, row_pad, col_pad):
    return pl.kernel(
        _gat_acc_body,
        mesh=_sc_mesh(),
        out_type=jax.ShapeDtypeStruct((ACC_PAD, C), jnp.float32),
        scratch_types=[
            pltpu.VMEM((AB,), jnp.int32),
            pltpu.VMEM((AB,), jnp.int32),
            pltpu.VMEM((AB,), jnp.float32),
            pltpu.VMEM((AB,), jnp.int32),
            pltpu.VMEM((AB, C), jnp.float32),
            pltpu.VMEM((AB,), jnp.int32),
            pltpu.VMEM((AB,), jnp.int32),
            pltpu.VMEM((AB,), jnp.float32),
            pltpu.VMEM((AB,), jnp.int32),
            pltpu.VMEM((AB, C), jnp.float32),
            pltpu.VMEM((SS, C), jnp.float32),
            pltpu.VMEM((SS, C), jnp.float32),
            pltpu.VMEM_SHARED((ACC_ROWS, C), jnp.float32),
            pltpu.SemaphoreType.DMA,
            pltpu.SemaphoreType.DMA,
        ],
    )(h, w_pad, row_pad, col_pad)


def _edge_gather_body(p_hbm, q_hbm, row_hbm, col_hbm, r_hbm,
                      idxr_v, idxc_v, prows_v, qrows_v, sem):
    wid = lax.axis_index("s") * 2 + lax.axis_index("c")
    base = wid * EPW

    def batch(g, _):
        off = base + g * EB
        pltpu.sync_copy(row_hbm.at[pl.ds(off, EB)], idxr_v)
        pltpu.sync_copy(col_hbm.at[pl.ds(off, EB)], idxc_v)
        pltpu.async_copy(p_hbm.at[idxr_v], prows_v, sem).wait()
        pltpu.async_copy(q_hbm.at[idxc_v], qrows_v, sem).wait()

        def addrow(r, _):
            for j in range(C // 16):
                sl = pl.ds(j * 16, 16)
                prows_v[r, sl] = prows_v[r, sl] + qrows_v[r, sl]
            return _

        lax.fori_loop(0, EB, addrow, None)
        pltpu.sync_copy(prows_v, r_hbm.at[pl.ds(off, EB)])
        return _

    lax.fori_loop(0, NBATCH, batch, None)


@functools.partial(jax.jit, static_argnames=())
def _edge_gather_add(p, q, row, col):
    return pl.kernel(
        _edge_gather_body,
        mesh=_sc_mesh(),
        out_type=jax.ShapeDtypeStruct((E, C), jnp.float32),
        scratch_types=[
            pltpu.VMEM((EB,), jnp.int32),
            pltpu.VMEM((EB,), jnp.int32),
            pltpu.VMEM((EB, C), jnp.float32),
            pltpu.VMEM((EB, C), jnp.float32),
            pltpu.SemaphoreType.DMA,
        ],
    )(p, q, row, col)

NODE_BLK = 2048
NH_PAD = ((N + NODE_BLK - 1) // NODE_BLK) * NODE_BLK
EDGE_BLK = 8000


def _heads_body(hjk_ref, wn1_ref, bn1_ref, wn2_ref, bn2_ref, node_ref):
    hjk = hjk_ref[...]
    t = jnp.maximum(hjk @ wn1_ref[...] + bn1_ref[...][None, :], 0.0)
    logits = t @ wn2_ref[...] + bn2_ref[...][None, :]
    m = jnp.max(logits, axis=1, keepdims=True)
    ex = jnp.exp(logits - m)
    node_ref[...] = ex / jnp.sum(ex, axis=1, keepdims=True)


def _node_head(hjk, Wn1, bn1, Wn2, bn2):
    hjk_p = jnp.pad(hjk, ((0, NH_PAD - N), (0, 0)))
    grid = (NH_PAD // NODE_BLK,)
    out = pl.pallas_call(
        _heads_body,
        grid=grid,
        in_specs=[
            pl.BlockSpec((NODE_BLK, C), lambda i: (i, 0)),
            pl.BlockSpec((C, C), lambda i: (0, 0)),
            pl.BlockSpec((C,), lambda i: (0,)),
            pl.BlockSpec((C, 2), lambda i: (0, 0)),
            pl.BlockSpec((2,), lambda i: (0,)),
        ],
        out_specs=pl.BlockSpec((NODE_BLK, 2), lambda i: (i, 0)),
        out_shape=jax.ShapeDtypeStruct((NH_PAD, 2), jnp.float32),
    )(hjk_p, Wn1, bn1, Wn2, bn2)
    return out[:N]


def _edge_head_body(r_ref, we2_ref, be2_ref, out_ref):
    t = jnp.maximum(r_ref[...], 0.0)
    logits = t @ we2_ref[...] + be2_ref[...][None, :]
    m = jnp.max(logits, axis=1, keepdims=True)
    ex = jnp.exp(logits - m)
    out_ref[...] = ex / jnp.sum(ex, axis=1, keepdims=True)


def _edge_head(r_rows, We2, be2):
    grid = (E // EDGE_BLK,)
    out = pl.pallas_call(
        _edge_head_body,
        grid=grid,
        in_specs=[
            pl.BlockSpec((EDGE_BLK, C), lambda i: (i, 0)),
            pl.BlockSpec((C, 2), lambda i: (0, 0)),
            pl.BlockSpec((2,), lambda i: (0,)),
        ],
        out_specs=pl.BlockSpec((EDGE_BLK, 2), lambda i: (i, 0)),
        out_shape=jax.ShapeDtypeStruct((E, 2), jnp.float32),
    )(r_rows, We2, be2)
    return out


def _gat_layer(x, row_pad, col_pad, W, a_src, a_dst, b):
    h = x @ W
    s_src = h @ a_src
    s_dst = h @ a_dst
    s_src_pad = jnp.pad(s_src, (0, N_PAD - N))
    s_dst_pad = jnp.pad(s_dst, (0, N_PAD - N))
    w = _edge_w(s_src_pad, s_dst_pad, row_pad, col_pad)
    w_pad = jnp.where(jnp.arange(E_PAD_E) < E, w, 0.0)
    num_pad = _gat_acc(h, w_pad, row_pad, col_pad)
    den_parts = _den_partials(w_pad, row_pad)
    num = num_pad[:N]
    den = den_parts[0, :N] + den_parts[1, :N]
    e_self = s_src + s_dst
    e_self = jnp.where(e_self > 0, e_self, 0.2 * e_self)
    w_self = jnp.exp(e_self)
    num = num + h * w_self[:, None]
    den = den + w_self
    return num / (den + 1e-16)[:, None] + b


def _lstm_dir(xs, Wih, Whh, bih, bhh, reverse):
    n = xs[0].shape[0]
    h = jnp.zeros((n, H), jnp.float32)
    c = jnp.zeros((n, H), jnp.float32)
    outs = [None, None]
    steps = [1, 0] if reverse else [0, 1]
    for t in steps:
        g = xs[t] @ Wih.T + h @ Whh.T + bih + bhh
        i, f, gg, o = jnp.split(g, 4, axis=-1)
        i = jax.nn.sigmoid(i)
        f = jax.nn.sigmoid(f)
        gg = jnp.tanh(gg)
        o = jax.nn.sigmoid(o)
        c = f * c + i * gg
        h = o * jnp.tanh(c)
        outs[t] = h
    return outs


def kernel(x, edge_index, W1, a_src1, a_dst1, b1, W2, a_src2, a_dst2, b2,
           Wih_f, Whh_f, bih_f, bhh_f, Wih_b, Whh_b, bih_b, bhh_b,
           Watt, batt, Wn1, bn1, Wn2, bn2, We1, be1, We2, be2):
    row = edge_index[0]
    col = edge_index[1]
    row_pad = jnp.pad(row, (0, E_PAD_E - E))
    col_pad = jnp.pad(col, (0, E_PAD_E - E))
    x1 = jax.nn.relu(_gat_layer(x, row_pad, col_pad, W1, a_src1, a_dst1, b1))
    x2 = jax.nn.relu(_gat_layer(x1, row_pad, col_pad, W2, a_src2, a_dst2, b2))
    xs = [x1, x2]
    hf = _lstm_dir(xs, Wih_f, Whh_f, bih_f, bhh_f, False)
    hb = _lstm_dir(xs, Wih_b, Whh_b, bih_b, bhh_b, True)
    wf = Watt[:H, 0]
    wb = Watt[H:, 0]
    att0 = hf[0] @ wf + hb[0] @ wb + batt[0]
    att1 = hf[1] @ wf + hb[1] @ wb + batt[0]
    m = jnp.maximum(att0, att1)
    a0 = jnp.exp(att0 - m)
    a1 = jnp.exp(att1 - m)
    s = a0 + a1
    hjk = x1 * (a0 / s)[:, None] + x2 * (a1 / s)[:, None]
    node_out = _node_head(hjk, Wn1, bn1, Wn2, bn2)
    p = hjk @ We1[:C] + be1
    q = hjk @ We1[C:]
    r_rows = _edge_gather_add(p, q, row, col)
    edge_out = _edge_head(r_rows, We2, be2)
    return (node_out, edge_out)
